# baseline (speedup 1.0000x reference)
import jax
import jax.numpy as jnp
from jax import lax
from jax.experimental import pallas as pl
from jax.experimental.pallas import tpu as pltpu
from jax.experimental.pallas import tpu_sc as plsc

_B = 16384
_D = 32
_NW = 32
_BPW = _B // _NW
_W = 8
_NWIN = _BPW // _W


def _fire(t_hbm, idx, blocks, sem, w):
    vec = idx[pl.ds(w * _W, 16)]
    cb = (vec >> 7) << 7
    for kk in range(_W):
        start = pl.multiple_of(cb[kk], 128)
        for d8 in range(4):
            pltpu.async_copy(
                t_hbm.at[d8, :, pl.ds(start, 128)],
                blocks.at[kk, pl.ds(d8 * 8, 8)], sem)


def _drain(t_hbm, blocks, sem):
    for kk in range(_W):
        for d8 in range(4):
            pltpu.make_async_copy(t_hbm.at[d8, :, pl.ds(0, 128)],
                                  blocks.at[kk, pl.ds(d8 * 8, 8)],
                                  sem).wait()


def _extract(idx, blocks, buf, w, lane16):
    vec = idx[pl.ds(w * _W, 16)]
    lanes = vec.at[lane16 & 7].get(mode="promise_in_bounds") & 127
    slots = lane16 & 7
    hi = lane16 >> 3
    flat = blocks.reshape(_W, _D * 128)
    for dp in range(_D // 2):
        drow = dp * 2 + hi
        vals = plsc.load_gather(flat, [slots, lanes + drow * 128])
        plsc.store_scatter(buf, [drow * _BPW + w * _W + slots], vals)


def _gather_table(t_hbm, idx, blk_a, blk_b, buf, sem_a, sem_b, lane16):
    _fire(t_hbm, idx, blk_a, sem_a, 0)

    def step(i, _):
        w_a, w_b = 2 * i, 2 * i + 1
        _fire(t_hbm, idx, blk_b, sem_b, w_b)
        _drain(t_hbm, blk_a, sem_a)
        _extract(idx, blk_a, buf, w_a, lane16)

        @pl.when(w_a + 2 < _NWIN)
        def _():
            _fire(t_hbm, idx, blk_a, sem_a, w_a + 2)

        _drain(t_hbm, blk_b, sem_b)
        _extract(idx, blk_b, buf, w_b, lane16)
        return 0

    lax.fori_loop(0, _NWIN // 2, step, 0)


def _body(users_hbm, items_hbm, ut_hbm, it_hbm, out_hbm,
          uidx, iidx, blk_a, blk_b, ubuf, ibuf, outv, sem_a, sem_b):
    wid = lax.axis_index("s") * 2 + lax.axis_index("c")
    base = wid * _BPW

    pltpu.sync_copy(users_hbm.at[pl.ds(base, _BPW)], uidx.at[pl.ds(0, _BPW)])
    pltpu.sync_copy(items_hbm.at[pl.ds(base, _BPW)], iidx.at[pl.ds(0, _BPW)])

    lane16 = lax.iota(jnp.int32, 16)
    _gather_table(ut_hbm, uidx, blk_a, blk_b, ubuf, sem_a, sem_b, lane16)
    _gather_table(it_hbm, iidx, blk_a, blk_b, ibuf, sem_a, sem_b, lane16)

    def block(g, _):
        sl0 = pl.ds(g * 16, 16)
        acc = ubuf[sl0] * ibuf[sl0]
        for d in range(1, _D):
            sl = pl.ds(d * _BPW + g * 16, 16)
            acc = acc + ubuf[sl] * ibuf[sl]
        outv[sl0] = 1.0 / (1.0 + jnp.exp(-acc))
        return 0

    lax.fori_loop(0, _BPW // 16, block, 0)

    pltpu.sync_copy(outv, out_hbm.at[pl.ds(base, _BPW)])


@jax.jit
def _run(users, items, user_table, item_table):
    mesh = plsc.VectorSubcoreMesh(core_axis_name="c", subcore_axis_name="s")
    k = pl.kernel(
        _body,
        out_type=jax.ShapeDtypeStruct((_B,), jnp.float32),
        mesh=mesh,
        scratch_types=[
            pltpu.VMEM((_BPW + 16,), jnp.int32),
            pltpu.VMEM((_BPW + 16,), jnp.int32),
            pltpu.VMEM((_W, _D, 128), jnp.float32),
            pltpu.VMEM((_W, _D, 128), jnp.float32),
            pltpu.VMEM((_D * _BPW,), jnp.float32),
            pltpu.VMEM((_D * _BPW,), jnp.float32),
            pltpu.VMEM((_BPW,), jnp.float32),
            pltpu.SemaphoreType.DMA,
            pltpu.SemaphoreType.DMA,
        ],
        compiler_params=pltpu.CompilerParams(
            use_tc_tiling_on_sc=True, needs_layout_passes=False),
    )
    return k(users, items,
             user_table.T.reshape(4, 8, 1000000),
             item_table.T.reshape(4, 8, 1000000))


def kernel(users, items, user_table, item_table):
    return _run(users, items, user_table, item_table)


# speedup vs baseline: 1.0085x; 1.0085x over previous
import jax
import jax.numpy as jnp
from jax import lax
from jax.experimental import pallas as pl
from jax.experimental.pallas import tpu as pltpu
from jax.experimental.pallas import tpu_sc as plsc

_B = 16384
_D = 32
_NW = 32
_BPW = _B // _NW
_W = 8
_NWIN = _BPW // _W


def _fire(t_hbm, idx, blocks, sem, w):
    vec = idx[pl.ds(w * _W, 16)]
    cb = (vec >> 7) << 7
    for kk in range(_W):
        start = pl.multiple_of(cb[kk], 128)
        pltpu.async_copy(t_hbm.at[:, pl.ds(start, 128)], blocks.at[kk], sem)


def _drain(t_hbm, blocks, sem):
    for kk in range(_W):
        pltpu.make_async_copy(t_hbm.at[:, pl.ds(0, 128)],
                              blocks.at[kk], sem).wait()


def _extract(idx, blocks, buf, w, lane16):
    vec = idx[pl.ds(w * _W, 16)]
    lanes = vec.at[lane16 & 7].get(mode="promise_in_bounds") & 127
    slots = lane16 & 7
    hi = lane16 >> 3
    flat = blocks.reshape(_W, _D * 128)
    for dp in range(_D // 2):
        drow = dp * 2 + hi
        vals = plsc.load_gather(flat, [slots, lanes + drow * 128])
        plsc.store_scatter(buf, [drow * _BPW + w * _W + slots], vals)


def _gather_table(t_hbm, idx, blk_a, blk_b, buf, sem_a, sem_b, lane16):
    _fire(t_hbm, idx, blk_a, sem_a, 0)

    def step(i, _):
        w_a, w_b = 2 * i, 2 * i + 1
        _fire(t_hbm, idx, blk_b, sem_b, w_b)
        _drain(t_hbm, blk_a, sem_a)
        _extract(idx, blk_a, buf, w_a, lane16)

        @pl.when(w_a + 2 < _NWIN)
        def _():
            _fire(t_hbm, idx, blk_a, sem_a, w_a + 2)

        _drain(t_hbm, blk_b, sem_b)
        _extract(idx, blk_b, buf, w_b, lane16)
        return 0

    lax.fori_loop(0, _NWIN // 2, step, 0)


def _body(users_hbm, items_hbm, ut_hbm, it_hbm, out_hbm,
          uidx, iidx, blk_a, blk_b, ubuf, ibuf, outv, sem_a, sem_b):
    wid = lax.axis_index("s") * 2 + lax.axis_index("c")
    base = wid * _BPW

    pltpu.sync_copy(users_hbm.at[pl.ds(base, _BPW)], uidx.at[pl.ds(0, _BPW)])
    pltpu.sync_copy(items_hbm.at[pl.ds(base, _BPW)], iidx.at[pl.ds(0, _BPW)])

    lane16 = lax.iota(jnp.int32, 16)
    _gather_table(ut_hbm, uidx, blk_a, blk_b, ubuf, sem_a, sem_b, lane16)
    _gather_table(it_hbm, iidx, blk_a, blk_b, ibuf, sem_a, sem_b, lane16)

    def block(g, _):
        sl0 = pl.ds(g * 16, 16)
        acc = ubuf[sl0] * ibuf[sl0]
        for d in range(1, _D):
            sl = pl.ds(d * _BPW + g * 16, 16)
            acc = acc + ubuf[sl] * ibuf[sl]
        outv[sl0] = 1.0 / (1.0 + jnp.exp(-acc))
        return 0

    lax.fori_loop(0, _BPW // 16, block, 0)

    pltpu.sync_copy(outv, out_hbm.at[pl.ds(base, _BPW)])


@jax.jit
def _run(users, items, user_table, item_table):
    mesh = plsc.VectorSubcoreMesh(core_axis_name="c", subcore_axis_name="s")
    k = pl.kernel(
        _body,
        out_type=jax.ShapeDtypeStruct((_B,), jnp.float32),
        mesh=mesh,
        scratch_types=[
            pltpu.VMEM((_BPW + 16,), jnp.int32),
            pltpu.VMEM((_BPW + 16,), jnp.int32),
            pltpu.VMEM((_W, _D, 128), jnp.float32),
            pltpu.VMEM((_W, _D, 128), jnp.float32),
            pltpu.VMEM((_D * _BPW,), jnp.float32),
            pltpu.VMEM((_D * _BPW,), jnp.float32),
            pltpu.VMEM((_BPW,), jnp.float32),
            pltpu.SemaphoreType.DMA,
            pltpu.SemaphoreType.DMA,
        ],
        compiler_params=pltpu.CompilerParams(
            use_tc_tiling_on_sc=True, needs_layout_passes=False),
    )
    return k(users, items, user_table.T, item_table.T)


def kernel(users, items, user_table, item_table):
    return _run(users, items, user_table, item_table)
